# baseline (speedup 1.0000x reference)
import functools

import jax
import jax.numpy as jnp
from jax.experimental import pallas as pl
from jax.experimental.pallas import tpu as pltpu

_NUM_CHUNKS = 8


def _round_up(x: int, m: int) -> int:
    return ((x + m - 1) // m) * m


def _gather_kernel(idx_ref, table_hbm, out_hbm, buf, gsems, wsem, *,
                   n_rows, num_chunks):
    chunk = n_rows // num_chunks
    d = buf.shape[1]
    for k in range(num_chunks):
        for r in range(k * chunk, (k + 1) * chunk):
            row = idx_ref[r]
            pltpu.make_async_copy(
                table_hbm.at[pl.ds(row, 1), : d // 2],
                buf.at[pl.ds(r, 1), : d // 2],
                gsems.at[k],
            ).start()
            pltpu.make_async_copy(
                table_hbm.at[pl.ds(row, 1), d // 2 :],
                buf.at[pl.ds(r, 1), d // 2 :],
                gsems.at[k],
            ).start()
    for k in range(num_chunks):
        pltpu.make_async_copy(
            table_hbm.at[pl.ds(0, chunk), :],
            buf.at[pl.ds(0, chunk), :],
            gsems.at[k],
        ).wait()
        pltpu.make_async_copy(
            buf.at[pl.ds(k * chunk, chunk), :],
            out_hbm.at[pl.ds(k * chunk, chunk), :],
            wsem,
        ).start()
    pltpu.make_async_copy(buf.at[...], out_hbm.at[...], wsem).wait()


def kernel(embedding_table, label_indices):
    nc, d = embedding_table.shape
    n = int(label_indices.shape[0])

    idx = jnp.clip(label_indices.astype(jnp.int32), 0, nc - 1)

    num_chunks = _NUM_CHUNKS
    n_pad = _round_up(max(n, 1), 8 * num_chunks)
    if n_pad != n:
        idx = jnp.pad(idx, (0, n_pad - n))

    gather_fn = functools.partial(_gather_kernel, n_rows=n_pad,
                                  num_chunks=num_chunks)
    grid_spec = pltpu.PrefetchScalarGridSpec(
        num_scalar_prefetch=1,
        grid=(1,),
        in_specs=[pl.BlockSpec(memory_space=pl.ANY)],
        out_specs=pl.BlockSpec(memory_space=pl.ANY),
        scratch_shapes=[
            pltpu.VMEM((n_pad, d), embedding_table.dtype),
            pltpu.SemaphoreType.DMA((num_chunks,)),
            pltpu.SemaphoreType.DMA,
        ],
    )
    out = pl.pallas_call(
        gather_fn,
        out_shape=jax.ShapeDtypeStruct((n_pad, d), embedding_table.dtype),
        grid_spec=grid_spec,
        compiler_params=pltpu.CompilerParams(
            dimension_semantics=("arbitrary",),
        ),
    )(idx, embedding_table)
    return out[:n]


# speedup vs baseline: 1.4462x; 1.4462x over previous
import functools

import jax
import jax.numpy as jnp
from jax.experimental import pallas as pl
from jax.experimental.pallas import tpu as pltpu


def _round_up(x: int, m: int) -> int:
    return ((x + m - 1) // m) * m


def _gather_kernel(idx_ref, table_hbm, out_ref, sem, *, block_rows, num_rows):
    for r in range(block_rows):
        row = jnp.minimum(jnp.maximum(idx_ref[r], 0), num_rows - 1)
        pltpu.make_async_copy(
            table_hbm.at[pl.ds(row, 1), :],
            out_ref.at[pl.ds(r, 1), :],
            sem,
        ).start()
    pltpu.make_async_copy(
        table_hbm.at[pl.ds(0, block_rows), :],
        out_ref.at[pl.ds(0, block_rows), :],
        sem,
    ).wait()


def kernel(embedding_table, label_indices):
    nc, d = embedding_table.shape
    n = int(label_indices.shape[0])

    idx = label_indices.astype(jnp.int32)
    n_pad = _round_up(max(n, 1), 8)
    if n_pad != n:
        idx = jnp.pad(idx, (0, n_pad - n))

    gather_fn = functools.partial(_gather_kernel, block_rows=n_pad,
                                  num_rows=nc)
    grid_spec = pltpu.PrefetchScalarGridSpec(
        num_scalar_prefetch=1,
        grid=(1,),
        in_specs=[pl.BlockSpec(memory_space=pl.ANY)],
        out_specs=pl.BlockSpec((n_pad, d), lambda i, idx_ref: (i, 0)),
        scratch_shapes=[pltpu.SemaphoreType.DMA],
    )
    out = pl.pallas_call(
        gather_fn,
        out_shape=jax.ShapeDtypeStruct((n_pad, d), embedding_table.dtype),
        grid_spec=grid_spec,
        compiler_params=pltpu.CompilerParams(
            dimension_semantics=("arbitrary",),
        ),
    )(idx, embedding_table)
    return out[:n]


# speedup vs baseline: 1.6037x; 1.1088x over previous
import functools

import jax
import jax.numpy as jnp
from jax.experimental import pallas as pl
from jax.experimental.pallas import tpu as pltpu

_NUM_CHUNKS = 2


def _round_up(x: int, m: int) -> int:
    return ((x + m - 1) // m) * m


def _gather_kernel(idx_ref, table_hbm, out_hbm, buf, gsems, wsem, *,
                   n_rows, num_chunks, num_table_rows):
    chunk = n_rows // num_chunks
    for k in range(num_chunks):
        for r in range(k * chunk, (k + 1) * chunk):
            row = jnp.minimum(jnp.maximum(idx_ref[r], 0), num_table_rows - 1)
            pltpu.make_async_copy(
                table_hbm.at[pl.ds(row, 1), :],
                buf.at[pl.ds(r, 1), :],
                gsems.at[k],
            ).start()
    for k in range(num_chunks):
        pltpu.make_async_copy(
            table_hbm.at[pl.ds(0, chunk), :],
            buf.at[pl.ds(0, chunk), :],
            gsems.at[k],
        ).wait()
        pltpu.make_async_copy(
            buf.at[pl.ds(k * chunk, chunk), :],
            out_hbm.at[pl.ds(k * chunk, chunk), :],
            wsem,
        ).start()
    pltpu.make_async_copy(buf.at[...], out_hbm.at[...], wsem).wait()


def kernel(embedding_table, label_indices):
    nc, d = embedding_table.shape
    n = int(label_indices.shape[0])

    idx = label_indices.astype(jnp.int32)
    num_chunks = _NUM_CHUNKS
    n_pad = _round_up(max(n, 1), 8 * num_chunks)
    if n_pad != n:
        idx = jnp.pad(idx, (0, n_pad - n))

    gather_fn = functools.partial(_gather_kernel, n_rows=n_pad,
                                  num_chunks=num_chunks, num_table_rows=nc)
    grid_spec = pltpu.PrefetchScalarGridSpec(
        num_scalar_prefetch=1,
        grid=(1,),
        in_specs=[pl.BlockSpec(memory_space=pl.ANY)],
        out_specs=pl.BlockSpec(memory_space=pl.ANY),
        scratch_shapes=[
            pltpu.VMEM((n_pad, d), embedding_table.dtype),
            pltpu.SemaphoreType.DMA((num_chunks,)),
            pltpu.SemaphoreType.DMA,
        ],
    )
    out = pl.pallas_call(
        gather_fn,
        out_shape=jax.ShapeDtypeStruct((n_pad, d), embedding_table.dtype),
        grid_spec=grid_spec,
        compiler_params=pltpu.CompilerParams(
            dimension_semantics=("arbitrary",),
        ),
    )(idx, embedding_table)
    return out[:n]


# speedup vs baseline: 1.6253x; 1.0135x over previous
import functools

import jax
import jax.numpy as jnp
from jax.experimental import pallas as pl
from jax.experimental.pallas import tpu as pltpu

_NUM_CHUNKS = 4


def _round_up(x: int, m: int) -> int:
    return ((x + m - 1) // m) * m


def _gather_kernel(idx_ref, table_hbm, out_hbm, buf, gsems, wsem, *,
                   n_rows, num_chunks, num_table_rows):
    chunk = n_rows // num_chunks
    for k in range(num_chunks):
        for r in range(k * chunk, (k + 1) * chunk):
            row = jnp.minimum(jnp.maximum(idx_ref[r], 0), num_table_rows - 1)
            pltpu.make_async_copy(
                table_hbm.at[pl.ds(row, 1), :],
                buf.at[pl.ds(r, 1), :],
                gsems.at[k],
            ).start()
    for k in range(num_chunks):
        pltpu.make_async_copy(
            table_hbm.at[pl.ds(0, chunk), :],
            buf.at[pl.ds(0, chunk), :],
            gsems.at[k],
        ).wait()
        pltpu.make_async_copy(
            buf.at[pl.ds(k * chunk, chunk), :],
            out_hbm.at[pl.ds(k * chunk, chunk), :],
            wsem,
        ).start()
    pltpu.make_async_copy(buf.at[...], out_hbm.at[...], wsem).wait()


def kernel(embedding_table, label_indices):
    nc, d = embedding_table.shape
    n = int(label_indices.shape[0])

    idx = label_indices.astype(jnp.int32)
    num_chunks = _NUM_CHUNKS
    n_pad = _round_up(max(n, 1), 8 * num_chunks)
    if n_pad != n:
        idx = jnp.pad(idx, (0, n_pad - n))

    gather_fn = functools.partial(_gather_kernel, n_rows=n_pad,
                                  num_chunks=num_chunks, num_table_rows=nc)
    grid_spec = pltpu.PrefetchScalarGridSpec(
        num_scalar_prefetch=1,
        grid=(1,),
        in_specs=[pl.BlockSpec(memory_space=pl.ANY)],
        out_specs=pl.BlockSpec(memory_space=pl.ANY),
        scratch_shapes=[
            pltpu.VMEM((n_pad, d), embedding_table.dtype),
            pltpu.SemaphoreType.DMA((num_chunks,)),
            pltpu.SemaphoreType.DMA,
        ],
    )
    out = pl.pallas_call(
        gather_fn,
        out_shape=jax.ShapeDtypeStruct((n_pad, d), embedding_table.dtype),
        grid_spec=grid_spec,
        compiler_params=pltpu.CompilerParams(
            dimension_semantics=("arbitrary",),
        ),
    )(idx, embedding_table)
    return out[:n]
